# Initial kernel scaffold; baseline (speedup 1.0000x reference)
import functools

import jax
import jax.numpy as jnp
from jax import lax
from jax.experimental import pallas as pl
from jax.experimental.pallas import tpu as pltpu
from jax.experimental.pallas import tpu_sc as plsc

VOCAB = 1000000
HIDDEN = 64
SEQ = 200
BSZ = 1024

NC = 2
NS = 16
L = 16
NW = NC * NS

ROWS = BSZ * SEQ
CHUNK = 128
CPW = ROWS // (NW * CHUNK)
POSREP = SEQ + CHUNK


def _emb_kernel(tok_hbm, ids_hbm, pos_hbm, out_hbm,
                ids_v, pos_v, in0, in1, out0, out1,
                gsem0, gsem1, osem0, osem1):
    wid = lax.axis_index("s") * NC + lax.axis_index("c")
    ins = (in0, in1)
    outs = (out0, out1)
    gsems = (gsem0, gsem1)
    osems = (osem0, osem1)

    pltpu.sync_copy(ids_hbm.at[pl.ds(wid * CPW, CPW)], ids_v)
    pltpu.sync_copy(pos_hbm.at[pl.ds(0, SEQ)], pos_v.at[pl.ds(0, SEQ)])
    pltpu.sync_copy(pos_hbm.at[pl.ds(0, CHUNK)], pos_v.at[pl.ds(SEQ, CHUNK)])

    def gather_start(g, j):
        pltpu.async_copy(tok_hbm.at[ids_v.at[g]], ins[j], gsems[j])

    def gather_wait(j):
        pltpu.make_async_copy(tok_hbm.at[pl.ds(0, CHUNK)], ins[j],
                              gsems[j]).wait()

    def scatter_start(g, j):
        base = (wid * CPW + g) * CHUNK
        pltpu.async_copy(outs[j], out_hbm.at[pl.ds(base, CHUNK)], osems[j])

    def scatter_wait(g, j):
        base = (wid * CPW + g) * CHUNK
        pltpu.make_async_copy(outs[j], out_hbm.at[pl.ds(base, CHUNK)],
                              osems[j]).wait()

    def add_pos(g, j):
        phi = lax.rem(g * CHUNK, SEQ)
        inb, outb = ins[j], outs[j]

        @pl.loop(0, CHUNK, step=2)
        def _(r):
            for rr in range(2):
                row = r + rr
                pr = phi + row
                for c in range(4):
                    s = pl.ds(c * L, L)
                    outb[row, s] = inb[row, s] + pos_v[pr, s]

    gather_start(0, 0)
    gather_start(1, 1)
    for g in (0, 1):
        j = g & 1
        gather_wait(j)
        add_pos(g, j)
        scatter_start(g, j)
        gather_start(g + 2, j)

    @pl.loop(2, CPW - 2, step=2)
    def _(g0):
        for jj in range(2):
            g = g0 + jj
            j = jj
            gather_wait(j)
            add_pos(g, j)
            scatter_wait(g - 2, j)
            scatter_start(g, j)
            gather_start(g + 2, j)

    for g in (CPW - 2, CPW - 1):
        j = g & 1
        gather_wait(j)
        add_pos(g, j)
        scatter_wait(g - 2, j)
        scatter_start(g, j)
    for g in (CPW - 2, CPW - 1):
        scatter_wait(g, g & 1)


@jax.jit
def _emb(tok_table, ids2d, pos_table):
    mesh = plsc.VectorSubcoreMesh(core_axis_name="c", subcore_axis_name="s")
    f = pl.kernel(
        _emb_kernel,
        out_type=jax.ShapeDtypeStruct((ROWS, HIDDEN), jnp.float32),
        mesh=mesh,
        scratch_types=[
            pltpu.VMEM((CPW, CHUNK), jnp.int32),
            pltpu.VMEM((POSREP, HIDDEN), jnp.float32),
            pltpu.VMEM((CHUNK, HIDDEN), jnp.float32),
            pltpu.VMEM((CHUNK, HIDDEN), jnp.float32),
            pltpu.VMEM((CHUNK, HIDDEN), jnp.float32),
            pltpu.VMEM((CHUNK, HIDDEN), jnp.float32),
            pltpu.SemaphoreType.DMA,
            pltpu.SemaphoreType.DMA,
            pltpu.SemaphoreType.DMA,
            pltpu.SemaphoreType.DMA,
        ],
    )
    return f(tok_table, ids2d, pos_table)


def kernel(input_ids, tok_table, pos_table):
    ids2d = input_ids.astype(jnp.int32).reshape(NW * CPW, CHUNK)
    out = _emb(tok_table, ids2d, pos_table)
    return out.reshape(BSZ, SEQ, HIDDEN)


# speedup vs baseline: 1.2259x; 1.2259x over previous
import functools

import jax
import jax.numpy as jnp
from jax import lax
from jax.experimental import pallas as pl
from jax.experimental.pallas import tpu as pltpu
from jax.experimental.pallas import tpu_sc as plsc

VOCAB = 1000000
HIDDEN = 64
SEQ = 200
BSZ = 1024

NC = 2
NS = 16
L = 16
NW = NC * NS

ROWS = BSZ * SEQ
CHUNK = 128
CPW = ROWS // (NW * CHUNK)
POSREP = SEQ + CHUNK


def _emb_kernel(tok_hbm, ids_hbm, pos_hbm, out_hbm,
                ids_v, pos_v, in0, in1, out0, out1,
                gsem0, gsem1, osem0, osem1):
    wid = lax.axis_index("s") * NC + lax.axis_index("c")
    ins = (in0, in1)
    outs = (out0, out1)
    gsems = (gsem0, gsem1)
    osems = (osem0, osem1)

    pltpu.sync_copy(ids_hbm.at[wid], ids_v)
    pltpu.sync_copy(pos_hbm.at[pl.ds(0, SEQ)], pos_v.at[pl.ds(0, SEQ)])
    pltpu.sync_copy(pos_hbm.at[pl.ds(0, CHUNK)], pos_v.at[pl.ds(SEQ, CHUNK)])

    def gather_start(g, j):
        pltpu.async_copy(tok_hbm.at[ids_v.at[g]], ins[j], gsems[j])

    def gather_wait(j):
        pltpu.make_async_copy(tok_hbm.at[pl.ds(0, CHUNK)], ins[j],
                              gsems[j]).wait()

    def scatter_start(g, j):
        base = (wid * CPW + g) * CHUNK
        pltpu.async_copy(outs[j], out_hbm.at[pl.ds(base, CHUNK)], osems[j])

    def scatter_wait(g, j):
        base = (wid * CPW + g) * CHUNK
        pltpu.make_async_copy(outs[j], out_hbm.at[pl.ds(base, CHUNK)],
                              osems[j]).wait()

    def add_pos(g, j):
        phi = lax.rem(g * CHUNK, SEQ)
        inb, outb = ins[j], outs[j]

        @pl.loop(0, CHUNK, step=2)
        def _(r):
            for rr in range(2):
                row = r + rr
                pr = phi + row
                for c in range(4):
                    s = pl.ds(c * L, L)
                    outb[row, s] = inb[row, s] + pos_v[pr, s]

    gather_start(0, 0)
    gather_start(1, 1)
    for g in (0, 1):
        j = g & 1
        gather_wait(j)
        add_pos(g, j)
        scatter_start(g, j)
        gather_start(g + 2, j)

    @pl.loop(2, CPW - 2, step=2)
    def _(g0):
        for jj in range(2):
            g = g0 + jj
            j = jj
            gather_wait(j)
            add_pos(g, j)
            scatter_wait(g - 2, j)
            scatter_start(g, j)
            gather_start(g + 2, j)

    for g in (CPW - 2, CPW - 1):
        j = g & 1
        gather_wait(j)
        add_pos(g, j)
        scatter_wait(g - 2, j)
        scatter_start(g, j)
    for g in (CPW - 2, CPW - 1):
        scatter_wait(g, g & 1)


@jax.jit
def _emb(tok_table, ids2d, pos_table):
    mesh = plsc.VectorSubcoreMesh(core_axis_name="c", subcore_axis_name="s")
    f = pl.kernel(
        _emb_kernel,
        out_type=jax.ShapeDtypeStruct((ROWS, HIDDEN), jnp.float32),
        mesh=mesh,
        compiler_params=pltpu.CompilerParams(use_tc_tiling_on_sc=False),
        scratch_types=[
            pltpu.VMEM((CPW, CHUNK), jnp.int32),
            pltpu.VMEM((POSREP, HIDDEN), jnp.float32),
            pltpu.VMEM((CHUNK, HIDDEN), jnp.float32),
            pltpu.VMEM((CHUNK, HIDDEN), jnp.float32),
            pltpu.VMEM((CHUNK, HIDDEN), jnp.float32),
            pltpu.VMEM((CHUNK, HIDDEN), jnp.float32),
            pltpu.SemaphoreType.DMA,
            pltpu.SemaphoreType.DMA,
            pltpu.SemaphoreType.DMA,
            pltpu.SemaphoreType.DMA,
        ],
    )
    return f(tok_table, ids2d, pos_table)


def kernel(input_ids, tok_table, pos_table):
    ids2d = input_ids.astype(jnp.int32).reshape(NW, CPW, CHUNK)
    out = _emb(tok_table, ids2d, pos_table)
    return out.reshape(BSZ, SEQ, HIDDEN)


# speedup vs baseline: 1.3592x; 1.1087x over previous
import jax
import jax.numpy as jnp
from jax import lax
from jax.experimental import pallas as pl
from jax.experimental.pallas import tpu as pltpu
from jax.experimental.pallas import tpu_sc as plsc

HIDDEN = 64
SEQ = 200
BSZ = 1024

NC = 2
NS = 16
L = 16
NW = NC * NS

SPW = BSZ // NW
G0, G1 = 128, SEQ - 128


def _emb_kernel(tok_hbm, ids_hbm, pos_hbm, out_hbm,
                ids_v, pos_v, in0, in1, out0, out1,
                gsem0, gsem1, osem0, osem1):
    wid = lax.axis_index("s") * NC + lax.axis_index("c")
    ins = (in0, in1)
    outs = (out0, out1)
    gsems = (gsem0, gsem1)
    osems = (osem0, osem1)

    pltpu.sync_copy(ids_hbm.at[pl.ds(wid * SPW, SPW)], ids_v)
    pltpu.sync_copy(pos_hbm.at[pl.ds(0, SEQ)], pos_v)

    def gather_start(s, j):
        pltpu.async_copy(tok_hbm.at[ids_v.at[s, pl.ds(0, G0)]],
                         ins[j].at[pl.ds(0, G0)], gsems[j])
        pltpu.async_copy(tok_hbm.at[ids_v.at[s, pl.ds(G0, G1)]],
                         ins[j].at[pl.ds(G0, G1)], gsems[j])

    def gather_wait(j):
        pltpu.make_async_copy(tok_hbm.at[pl.ds(0, SEQ)], ins[j],
                              gsems[j]).wait()

    def scatter_start(s, j):
        pltpu.async_copy(outs[j], out_hbm.at[wid * SPW + s], osems[j])

    def scatter_wait(s, j):
        pltpu.make_async_copy(outs[j], out_hbm.at[wid * SPW + s],
                              osems[j]).wait()

    def add_pos(j):
        inb, outb = ins[j], outs[j]

        @pl.loop(0, SEQ, step=2)
        def _(r):
            for rr in range(2):
                row = r + rr
                for c in range(4):
                    sl = pl.ds(c * L, L)
                    outb[row, sl] = inb[row, sl] + pos_v[row, sl]

    gather_start(0, 0)
    gather_start(1, 1)
    for s in (0, 1):
        j = s & 1
        gather_wait(j)
        add_pos(j)
        scatter_start(s, j)
        gather_start(s + 2, j)

    @pl.loop(2, SPW - 2, step=2)
    def _(s0):
        for jj in range(2):
            s = s0 + jj
            gather_wait(jj)
            add_pos(jj)
            scatter_wait(s - 2, jj)
            scatter_start(s, jj)
            gather_start(s + 2, jj)

    for s in (SPW - 2, SPW - 1):
        j = s & 1
        gather_wait(j)
        add_pos(j)
        scatter_wait(s - 2, j)
        scatter_start(s, j)
    for s in (SPW - 2, SPW - 1):
        scatter_wait(s, s & 1)


@jax.jit
def _emb(tok_table, ids, pos_table):
    mesh = plsc.VectorSubcoreMesh(core_axis_name="c", subcore_axis_name="s")
    f = pl.kernel(
        _emb_kernel,
        out_type=jax.ShapeDtypeStruct((BSZ, SEQ, HIDDEN), jnp.float32),
        mesh=mesh,
        compiler_params=pltpu.CompilerParams(use_tc_tiling_on_sc=False),
        scratch_types=[
            pltpu.VMEM((SPW, SEQ), jnp.int32),
            pltpu.VMEM((SEQ, HIDDEN), jnp.float32),
            pltpu.VMEM((SEQ, HIDDEN), jnp.float32),
            pltpu.VMEM((SEQ, HIDDEN), jnp.float32),
            pltpu.VMEM((SEQ, HIDDEN), jnp.float32),
            pltpu.VMEM((SEQ, HIDDEN), jnp.float32),
            pltpu.SemaphoreType.DMA,
            pltpu.SemaphoreType.DMA,
            pltpu.SemaphoreType.DMA,
            pltpu.SemaphoreType.DMA,
        ],
    )
    return f(tok_table, ids, pos_table)


def kernel(input_ids, tok_table, pos_table):
    return _emb(tok_table, input_ids.astype(jnp.int32), pos_table)
